# baseline (speedup 1.0000x reference)
import functools

import jax
import jax.numpy as jnp
from jax.experimental import pallas as pl
from jax.experimental.pallas import tpu as pltpu

_TILE = 256
_GROUP = 8


def _gcn_kernel(x_ref, a_ref, ws1_ref, wn1_ref, ws2_ref, wn2_ref,
                wa1_ref, wb1_ref, wf_ref, b1_ref, b2_ref,
                ba1_ref, bb1_ref, bf_ref, o_ref):
    t, n, f_in = x_ref.shape
    r = t * n
    p = _GROUP * n
    n_groups = t // _GROUP

    xb = x_ref[...].reshape(r, f_in)
    a = a_ref[:, :, :n].reshape(r, n).astype(jnp.float32)

    w1 = jnp.concatenate([ws1_ref[...], wn1_ref[...]], axis=1)
    w2 = jnp.concatenate([ws2_ref[...], wn2_ref[...]], axis=1)
    wh = jnp.concatenate([wa1_ref[...], wb1_ref[...]], axis=1)
    bh = jnp.concatenate([ba1_ref[...], bb1_ref[...]], axis=1)

    deg = jnp.sum(a, axis=1, keepdims=True)
    rnorm = jnp.where(deg > 0.0, 1.0 / deg, 0.0)
    an = a * rnorm

    rows = jax.lax.broadcasted_iota(jnp.int32, (p, p), 0) // n
    cols = jax.lax.broadcasted_iota(jnp.int32, (p, p), 1) // n
    same = (rows == cols).astype(jnp.float32)

    tj = jax.lax.broadcasted_iota(jnp.int32, (n, p), 0)
    tc = jax.lax.broadcasted_iota(jnp.int32, (n, p), 1)
    tile_mat = (tc % n == tj).astype(jnp.float32)

    blocks = []
    for gi in range(n_groups):
        g = an[gi * p:(gi + 1) * p, :]
        rep = jnp.dot(g, tile_mat, preferred_element_type=jnp.float32)
        blocks.append(rep * same)

    def agg(feat):
        outs = [jnp.dot(blocks[gi], feat[gi * p:(gi + 1) * p, :],
                        preferred_element_type=jnp.float32)
                for gi in range(n_groups)]
        return jnp.concatenate(outs, axis=0)

    y1 = jnp.dot(xb, w1, preferred_element_type=jnp.float32)
    n1 = agg(y1[:, f_in:])
    h1 = jnp.maximum(y1[:, :f_in] + n1 + b1_ref[...], 0.0)

    y2 = jnp.dot(h1, w2, preferred_element_type=jnp.float32)
    n2 = agg(y2[:, f_in:])
    h2 = jnp.maximum(y2[:, :f_in] + n2 + b2_ref[...], 0.0)

    u = jnp.dot(h2, wh, preferred_element_type=jnp.float32) + bh
    ub = jnp.maximum(u, 0.0)
    fin_t = jax.lax.dot_general(
        wf_ref[...], ub, (((1,), (1,)), ((), ())),
        preferred_element_type=jnp.float32) + bf_ref[...]
    o_ref[...] = fin_t.reshape(3, t, n)


@functools.partial(jax.jit, static_argnames=())
def _forward(x, adj_p, ws1, wn1, ws2, wn2, wa1, wb1, wf, b1, b2, ba1, bb1, bf):
    b, n, f_in = x.shape
    t = _TILE
    full = lambda w: pl.BlockSpec(w.shape, lambda i: tuple(0 for _ in w.shape))
    return pl.pallas_call(
        _gcn_kernel,
        out_shape=jax.ShapeDtypeStruct((3, b, n), jnp.float32),
        grid=(b // t,),
        in_specs=[
            pl.BlockSpec((t, n, f_in), lambda i: (i, 0, 0)),
            pl.BlockSpec((t, n, adj_p.shape[2]), lambda i: (i, 0, 0)),
        ] + [full(w) for w in (ws1, wn1, ws2, wn2, wa1, wb1, wf,
                               b1, b2, ba1, bb1, bf)],
        out_specs=pl.BlockSpec((3, t, n), lambda i: (0, i, 0)),
        compiler_params=pltpu.CompilerParams(
            dimension_semantics=("parallel",),
            vmem_limit_bytes=64 * 1024 * 1024,
        ),
    )(x, adj_p, ws1, wn1, ws2, wn2, wa1, wb1, wf, b1, b2, ba1, bb1, bf)


def kernel(x, adj, ws1, wn1, b1, ws2, wn2, b2,
           wa1, ba1, wa2, ba2, wb1, bb1, wb2, bb2):
    b, n, f_in = x.shape
    h = wa2.shape[0]
    zf = jnp.zeros((h, 1), jnp.float32)
    zt = jnp.zeros((h, 2), jnp.float32)
    wf = jnp.concatenate([
        jnp.concatenate([wa2, zf], axis=1),
        jnp.concatenate([zt, wb2], axis=1),
    ], axis=0).T
    bf_ = jnp.concatenate([ba2, bb2], axis=1).T
    adj_p = jnp.pad(adj.astype(jnp.bfloat16), ((0, 0), (0, 0), (0, f_in - n)))
    fin_t = _forward(x, adj_p, ws1, wn1, ws2, wn2, wa1, wb1, wf,
                     b1, b2, ba1, bb1, bf_)
    oa = jnp.transpose(fin_t[0:2], (1, 2, 0))
    ob = jnp.transpose(fin_t[2:3], (1, 2, 0))
    return {"am1-charges": oa, "am1-wbo-like": ob}


# speedup vs baseline: 1.1356x; 1.1356x over previous
import functools

import jax
import jax.numpy as jnp
from jax.experimental import pallas as pl
from jax.experimental.pallas import tpu as pltpu

_TILE = 256
_GROUP = 8


def _gcn_kernel(x_ref, a_ref, ws1_ref, wn1_ref, ws2_ref, wn2_ref,
                wa1_ref, wb1_ref, wf_ref, b1_ref, b2_ref,
                ba1_ref, bb1_ref, bf_ref, o_ref):
    t, n, f_in = x_ref.shape
    r = t * n
    p = _GROUP * n
    n_groups = t // _GROUP

    xb = x_ref[...].reshape(r, f_in)
    a = a_ref[...].reshape(r, n).astype(jnp.float32)

    w1 = jnp.concatenate([ws1_ref[...], wn1_ref[...]], axis=1)
    w2 = jnp.concatenate([ws2_ref[...], wn2_ref[...]], axis=1)
    wh = jnp.concatenate([wa1_ref[...], wb1_ref[...]], axis=1)
    bh = jnp.concatenate([ba1_ref[...], bb1_ref[...]], axis=1)

    deg = jnp.sum(a, axis=1, keepdims=True)
    rnorm = jnp.where(deg > 0.0, 1.0 / deg, 0.0)
    an = a * rnorm

    rows = jax.lax.broadcasted_iota(jnp.int32, (p, p), 0) // n
    cols = jax.lax.broadcasted_iota(jnp.int32, (p, p), 1) // n
    same = (rows == cols).astype(jnp.float32)

    tj = jax.lax.broadcasted_iota(jnp.int32, (n, p), 0)
    tc = jax.lax.broadcasted_iota(jnp.int32, (n, p), 1)
    tile_mat = (tc % n == tj).astype(jnp.float32)

    blocks = []
    for gi in range(n_groups):
        g = an[gi * p:(gi + 1) * p, :]
        rep = jnp.dot(g, tile_mat, preferred_element_type=jnp.float32)
        blocks.append(rep * same)

    def agg(feat):
        outs = [jnp.dot(blocks[gi], feat[gi * p:(gi + 1) * p, :],
                        preferred_element_type=jnp.float32)
                for gi in range(n_groups)]
        return jnp.concatenate(outs, axis=0)

    y1 = jnp.dot(xb, w1, preferred_element_type=jnp.float32)
    n1 = agg(y1[:, f_in:])
    h1 = jnp.maximum(y1[:, :f_in] + n1 + b1_ref[...], 0.0)

    y2 = jnp.dot(h1, w2, preferred_element_type=jnp.float32)
    n2 = agg(y2[:, f_in:])
    h2 = jnp.maximum(y2[:, :f_in] + n2 + b2_ref[...], 0.0)

    u = jnp.dot(h2, wh, preferred_element_type=jnp.float32) + bh
    ub = jnp.maximum(u, 0.0)
    fin_t = jax.lax.dot_general(
        wf_ref[...], ub, (((1,), (1,)), ((), ())),
        preferred_element_type=jnp.float32) + bf_ref[...]
    o_ref[...] = fin_t.reshape(3, t, n)


@functools.partial(jax.jit, static_argnames=())
def _forward(x, adj_p, ws1, wn1, ws2, wn2, wa1, wb1, wf, b1, b2, ba1, bb1, bf):
    b, n, f_in = x.shape
    t = _TILE
    full = lambda w: pl.BlockSpec(w.shape, lambda i: tuple(0 for _ in w.shape))
    return pl.pallas_call(
        _gcn_kernel,
        out_shape=jax.ShapeDtypeStruct((3, b, n), jnp.float32),
        grid=(b // t,),
        in_specs=[
            pl.BlockSpec((t, n, f_in), lambda i: (i, 0, 0)),
            pl.BlockSpec((t, n, adj_p.shape[2]), lambda i: (i, 0, 0)),
        ] + [full(w) for w in (ws1, wn1, ws2, wn2, wa1, wb1, wf,
                               b1, b2, ba1, bb1, bf)],
        out_specs=pl.BlockSpec((3, t, n), lambda i: (0, i, 0)),
        compiler_params=pltpu.CompilerParams(
            dimension_semantics=("parallel",),
            vmem_limit_bytes=64 * 1024 * 1024,
        ),
    )(x, adj_p, ws1, wn1, ws2, wn2, wa1, wb1, wf, b1, b2, ba1, bb1, bf)


def kernel(x, adj, ws1, wn1, b1, ws2, wn2, b2,
           wa1, ba1, wa2, ba2, wb1, bb1, wb2, bb2):
    b, n, f_in = x.shape
    h = wa2.shape[0]
    zf = jnp.zeros((h, 1), jnp.float32)
    zt = jnp.zeros((h, 2), jnp.float32)
    wf = jnp.concatenate([
        jnp.concatenate([wa2, zf], axis=1),
        jnp.concatenate([zt, wb2], axis=1),
    ], axis=0).T
    bf_ = jnp.concatenate([ba2, bb2], axis=1).T
    adj_p = adj.astype(jnp.bfloat16)
    fin_t = _forward(x, adj_p, ws1, wn1, ws2, wn2, wa1, wb1, wf,
                     b1, b2, ba1, bb1, bf_)
    oa = jnp.transpose(fin_t[0:2], (1, 2, 0))
    ob = jnp.transpose(fin_t[2:3], (1, 2, 0))
    return {"am1-charges": oa, "am1-wbo-like": ob}
